# baseline (speedup 1.0000x reference)
import functools
import jax
import jax.numpy as jnp
from jax import lax
from jax.experimental import pallas as pl
from jax.experimental.pallas import tpu as pltpu
from jax.experimental.pallas import tpu_sc as plsc

NC, NS, L = 2, 16, 16
NW = NC * NS
D = 64
BATCH, SEQ = 4096, 200
SPW = BATCH // NW
C = SEQ
K = 2
CK = C // K
NBUF = 2

_mesh = plsc.VectorSubcoreMesh(core_axis_name="c", subcore_axis_name="s")


@functools.partial(
    pl.kernel,
    out_type=jax.ShapeDtypeStruct((BATCH, SEQ, D), jnp.float32),
    mesh=_mesh,
    scratch_types=[
        pltpu.VMEM((NBUF, K, CK), jnp.int32),
        pltpu.VMEM((NBUF, C, D), jnp.float32),
        pltpu.VMEM((NBUF, C, D), jnp.float32),
        pltpu.VMEM((C, D), jnp.float32),
        pltpu.SemaphoreType.DMA,
        pltpu.SemaphoreType.DMA,
        pltpu.SemaphoreType.DMA,
    ],
    compiler_params=pltpu.CompilerParams(use_tc_tiling_on_sc=False),
)
def _emb_kernel(tok_hbm, table_hbm, pe_hbm, out_hbm,
                idx_v, rows_v, out_v, pe_v, gsem, ssem, isem):
    wid = lax.axis_index("s") * NC + lax.axis_index("c")
    base = wid * SPW
    pltpu.sync_copy(pe_hbm.at[pl.ds(0, C)], pe_v)

    def fire_gathers(b):
        for j in range(K):
            pltpu.async_copy(table_hbm.at[idx_v.at[b, j]],
                             rows_v.at[b, pl.ds(j * CK, CK)], gsem)

    for b in range(NBUF):
        pltpu.sync_copy(tok_hbm.at[base + b], idx_v.at[b])
        fire_gathers(b)

    def outer(t, _):
        for b in range(NBUF):
            g = t * NBUF + b
            pltpu.make_async_copy(
                table_hbm.at[pl.ds(0, C)], rows_v.at[b], gsem).wait()

            @pl.when(g + NBUF < SPW)
            def _():
                pltpu.async_copy(tok_hbm.at[base + g + NBUF], idx_v.at[b],
                                 isem)

            @pl.when(g >= NBUF)
            def _():
                pltpu.make_async_copy(
                    out_v.at[b], out_hbm.at[base], ssem).wait()

            rb, ob = rows_v.at[b], out_v.at[b]

            def row(i, _):
                for v in range(D // L):
                    sl = pl.ds(v * L, L)
                    ob[i, sl] = rb[i, sl] * 8.0 + pe_v[i, sl]
                return 0
            lax.fori_loop(0, C, row, 0)

            pltpu.async_copy(out_v.at[b], out_hbm.at[base + g], ssem)

            @pl.when(g + NBUF < SPW)
            def _():
                pltpu.make_async_copy(
                    tok_hbm.at[base], idx_v.at[b], isem).wait()
                fire_gathers(b)
        return 0

    lax.fori_loop(0, SPW // NBUF, outer, 0)

    for b in range(NBUF):
        pltpu.make_async_copy(out_v.at[b], out_hbm.at[base], ssem).wait()


def kernel(token_sequences, embedding_weight, positional_embedding):
    tok = token_sequences.reshape(BATCH, K, CK)
    pe = positional_embedding.reshape(positional_embedding.shape[1], D)
    return _emb_kernel(tok, embedding_weight, pe)
